# baseline (speedup 1.0000x reference)
import functools

import jax
import jax.numpy as jnp
from jax import lax
from jax.experimental import pallas as pl
from jax.experimental.pallas import tpu as pltpu
from jax.experimental.pallas import tpu_sc as plsc

N = 10000
D = 128
DA = 144
E = 320000
NC = 2
NS = 16
NW = NC * NS
EPW = E // NW
CH = 80
NCHUNK = EPW // CH
SEG = 25
RPT = 640
RCH = 80

_mesh = plsc.VectorSubcoreMesh(core_axis_name="c", subcore_axis_name="s")


def _seg_sum_body(width, *refs):
    (src_hbm, dst_hbm, feat_hbm, zrow_hbm, sum_out,
     sidx0_v, didx0_v, sidx1_v, didx1_v, rows0_v, rows1_v, acc_sh,
     sem0, sem1, semi) = refs

    c = lax.axis_index("c")
    s = lax.axis_index("s")
    wid = c * NS + s

    r0 = s * RPT
    nch = jnp.where(s < NS - 1, RPT // RCH, (N - (NS - 1) * RPT) // RCH)

    pltpu.async_copy(src_hbm.at[pl.ds(wid * NCHUNK, SEG)], sidx0_v, semi)
    pltpu.async_copy(dst_hbm.at[pl.ds(wid * NCHUNK, SEG)], didx0_v, semi)

    def zinit(i, carry):
        pltpu.async_copy(zrow_hbm, acc_sh.at[pl.ds(r0 + i * RCH, RCH)], sem1)
        return carry

    lax.fori_loop(0, nch, zinit, 0)

    def zdrain(i, carry):
        pltpu.make_async_copy(zrow_hbm, acc_sh.at[pl.ds(r0, RCH)], sem1).wait()
        return carry

    lax.fori_loop(0, nch, zdrain, 0)
    _wait_idx(src_hbm, sidx0_v, didx0_v, semi)
    plsc.subcore_barrier()

    bufs = (rows0_v, rows1_v)
    sems = (sem0, sem1)
    idxb = ((sidx0_v, didx0_v), (sidx1_v, didx1_v))
    NSEG = NCHUNK // SEG

    def start_gather(sidx, k, b):
        pltpu.async_copy(feat_hbm.at[sidx.at[k]], bufs[b], sems[b])

    def wait_gather(sidx, b):
        pltpu.make_async_copy(feat_hbm.at[sidx.at[0]], bufs[b], sems[b]).wait()

    def scatter(didx, k, b):
        pltpu.sync_copy(bufs[b], acc_sh.at[didx.at[k]], add=True)

    for seg in range(NSEG):
        sidx, didx = idxb[seg % 2]
        if seg + 1 < NSEG:
            nsidx, ndidx = idxb[(seg + 1) % 2]
            row = wid * NCHUNK + (seg + 1) * SEG
            pltpu.async_copy(src_hbm.at[pl.ds(row, SEG)], nsidx, semi)
            pltpu.async_copy(dst_hbm.at[pl.ds(row, SEG)], ndidx, semi)
        start_gather(sidx, 0, 0)

        def pair(g, c2, sidx=sidx, didx=didx):
            k0 = 2 * g
            start_gather(sidx, k0 + 1, 1)
            wait_gather(sidx, 0)
            scatter(didx, k0, 0)
            start_gather(sidx, k0 + 2, 0)
            wait_gather(sidx, 1)
            scatter(didx, k0 + 1, 1)
            return c2

        lax.fori_loop(0, (SEG - 1) // 2, pair, 0)
        wait_gather(sidx, 0)
        scatter(didx, SEG - 1, 0)
        if seg + 1 < NSEG:
            _wait_idx(src_hbm, nsidx, ndidx, semi)

    plsc.subcore_barrier()

    def writeout(i, carry):
        off = r0 + i * RCH
        pltpu.async_copy(acc_sh.at[pl.ds(off, RCH)],
                         sum_out.at[c, pl.ds(off, RCH)], sem0)
        return carry

    lax.fori_loop(0, nch, writeout, 0)

    def wdrain(i, carry):
        pltpu.make_async_copy(acc_sh.at[pl.ds(r0, RCH)],
                              sum_out.at[c, pl.ds(r0, RCH)], sem0).wait()
        return carry

    lax.fori_loop(0, nch, wdrain, 0)


def _wait_idx(src_hbm, sidx, didx, semi):
    pltpu.make_async_copy(src_hbm.at[pl.ds(0, SEG)], sidx, semi).wait()
    pltpu.make_async_copy(src_hbm.at[pl.ds(0, SEG)], didx, semi).wait()


def _make_seg_sum(width):
    return pl.kernel(
        functools.partial(_seg_sum_body, width),
        out_type=[jax.ShapeDtypeStruct((NC, N, width), jnp.float32)],
        mesh=_mesh,
        compiler_params=pltpu.CompilerParams(use_tc_tiling_on_sc=False),
        scratch_types=[
            pltpu.VMEM((SEG, CH), jnp.int32),
            pltpu.VMEM((SEG, CH), jnp.int32),
            pltpu.VMEM((SEG, CH), jnp.int32),
            pltpu.VMEM((SEG, CH), jnp.int32),
            pltpu.VMEM((CH, width), jnp.float32),
            pltpu.VMEM((CH, width), jnp.float32),
            pltpu.VMEM_SHARED((N, width), jnp.float32),
            pltpu.SemaphoreType.DMA,
            pltpu.SemaphoreType.DMA,
            pltpu.SemaphoreType.DMA,
        ],
    )


_seg_sum_aug = _make_seg_sum(DA)
_seg_sum_plain = _make_seg_sum(D)


TB = 1000
GRID = N // TB


def _tc1_body(s_ref, x_ref, wn_ref, ws_ref, b_ref, o_ref, r_ref):
    ssum = s_ref[0] + s_ref[1]
    cnt = ssum[:, D:D + 1]
    recip = 1.0 / jnp.maximum(cnt, 1.0)
    mean = ssum[:, :D] * recip
    y = (jnp.dot(mean, wn_ref[...], preferred_element_type=jnp.float32)
         + jnp.dot(x_ref[...], ws_ref[...], preferred_element_type=jnp.float32)
         + b_ref[...])
    o_ref[...] = jnp.maximum(y, 0.0)
    r_ref[...] = jnp.broadcast_to(recip, (TB, 8))


def _tc2_body(s_ref, r_ref, h_ref, wn_ref, ws_ref, b_ref, o_ref):
    ssum = s_ref[0] + s_ref[1]
    mean = ssum * r_ref[:, 0:1]
    o_ref[...] = (
        jnp.dot(mean, wn_ref[...], preferred_element_type=jnp.float32)
        + jnp.dot(h_ref[...], ws_ref[...], preferred_element_type=jnp.float32)
        + b_ref[...])


_tc1 = pl.pallas_call(
    _tc1_body,
    grid=(GRID,),
    in_specs=[
        pl.BlockSpec((NC, TB, DA), lambda i: (0, i, 0)),
        pl.BlockSpec((TB, D), lambda i: (i, 0)),
        pl.BlockSpec((D, D), lambda i: (0, 0)),
        pl.BlockSpec((D, D), lambda i: (0, 0)),
        pl.BlockSpec((1, D), lambda i: (0, 0)),
    ],
    out_specs=[
        pl.BlockSpec((TB, D), lambda i: (i, 0)),
        pl.BlockSpec((TB, 8), lambda i: (i, 0)),
    ],
    out_shape=[
        jax.ShapeDtypeStruct((N, D), jnp.float32),
        jax.ShapeDtypeStruct((N, 8), jnp.float32),
    ],
)

_tc2 = pl.pallas_call(
    _tc2_body,
    grid=(GRID,),
    in_specs=[
        pl.BlockSpec((NC, TB, D), lambda i: (0, i, 0)),
        pl.BlockSpec((TB, 8), lambda i: (i, 0)),
        pl.BlockSpec((TB, D), lambda i: (i, 0)),
        pl.BlockSpec((D, D), lambda i: (0, 0)),
        pl.BlockSpec((D, D), lambda i: (0, 0)),
        pl.BlockSpec((1, D), lambda i: (0, 0)),
    ],
    out_specs=pl.BlockSpec((TB, D), lambda i: (i, 0)),
    out_shape=jax.ShapeDtypeStruct((N, D), jnp.float32),
)


def kernel(node_features, edge_index, W1_neigh, W1_self, b1,
           W2_neigh, W2_self, b2):
    src = edge_index[0].reshape(E // CH, CH)
    dst = edge_index[1].reshape(E // CH, CH)
    ones_pad = jnp.concatenate(
        [jnp.ones((N, 1), jnp.float32), jnp.zeros((N, DA - D - 1), jnp.float32)],
        axis=1)
    x_aug = jnp.concatenate([node_features, ones_pad], axis=1)
    zrow_a = jnp.zeros((RCH, DA), jnp.float32)
    zrow = jnp.zeros((RCH, D), jnp.float32)

    (sums1,) = _seg_sum_aug(src, dst, x_aug, zrow_a)
    h, rcp = _tc1(sums1, node_features, W1_neigh.T, W1_self.T,
                  b1.reshape(1, D))
    (sums2,) = _seg_sum_plain(src, dst, h, zrow)
    out = _tc2(sums2, rcp, h, W2_neigh.T, W2_self.T, b2.reshape(1, D))
    return out


# speedup vs baseline: 1.0723x; 1.0723x over previous
import functools

import jax
import jax.numpy as jnp
from jax import lax
from jax.experimental import pallas as pl
from jax.experimental.pallas import tpu as pltpu
from jax.experimental.pallas import tpu_sc as plsc

N = 10000
D = 128
DA = 144
E = 320000
NC = 2
NS = 16
NW = NC * NS
EPW = E // NW
CH = 80
NCHUNK = EPW // CH
SEG = 25
RPT = 640
RCH = 80

_mesh = plsc.VectorSubcoreMesh(core_axis_name="c", subcore_axis_name="s")


def _seg_sum_body(width, *refs):
    (src_hbm, dst_hbm, feat_hbm, zrow_hbm, sum_out,
     sidx0_v, didx0_v, sidx1_v, didx1_v, rows0_v, rows1_v, acc_sh,
     sem0, sem1, semi) = refs

    c = lax.axis_index("c")
    s = lax.axis_index("s")
    wid = c * NS + s

    r0 = s * RPT
    nch = jnp.where(s < NS - 1, RPT // RCH, (N - (NS - 1) * RPT) // RCH)

    pltpu.async_copy(src_hbm.at[pl.ds(wid * NCHUNK, SEG)], sidx0_v, semi)
    pltpu.async_copy(dst_hbm.at[pl.ds(wid * NCHUNK, SEG)], didx0_v, semi)

    pltpu.sync_copy(zrow_hbm, rows0_v)

    def zinit(i, carry):
        pltpu.sync_copy(rows0_v, acc_sh.at[pl.ds(r0 + i * RCH, RCH)])
        return carry

    lax.fori_loop(0, nch, zinit, 0)
    _wait_idx(src_hbm, sidx0_v, didx0_v, semi)
    plsc.subcore_barrier()

    bufs = (rows0_v, rows1_v)
    sems = (sem0, sem1)
    idxb = ((sidx0_v, didx0_v), (sidx1_v, didx1_v))
    NSEG = NCHUNK // SEG

    def start_gather(sidx, k, b):
        pltpu.async_copy(feat_hbm.at[sidx.at[k]], bufs[b], sems[b])

    def wait_gather(sidx, b):
        pltpu.make_async_copy(feat_hbm.at[sidx.at[0]], bufs[b], sems[b]).wait()

    def scatter(didx, k, b):
        pltpu.sync_copy(bufs[b], acc_sh.at[didx.at[k]], add=True)

    for seg in range(NSEG):
        sidx, didx = idxb[seg % 2]
        if seg + 1 < NSEG:
            nsidx, ndidx = idxb[(seg + 1) % 2]
            row = wid * NCHUNK + (seg + 1) * SEG
            pltpu.async_copy(src_hbm.at[pl.ds(row, SEG)], nsidx, semi)
            pltpu.async_copy(dst_hbm.at[pl.ds(row, SEG)], ndidx, semi)
        start_gather(sidx, 0, 0)

        def pair(g, c2, sidx=sidx, didx=didx):
            k0 = 2 * g
            start_gather(sidx, k0 + 1, 1)
            wait_gather(sidx, 0)
            scatter(didx, k0, 0)
            start_gather(sidx, k0 + 2, 0)
            wait_gather(sidx, 1)
            scatter(didx, k0 + 1, 1)
            return c2

        lax.fori_loop(0, (SEG - 1) // 2, pair, 0)
        wait_gather(sidx, 0)
        scatter(didx, SEG - 1, 0)
        if seg + 1 < NSEG:
            _wait_idx(src_hbm, nsidx, ndidx, semi)

    plsc.subcore_barrier()

    def writeout(i, carry):
        off = r0 + i * RCH
        pltpu.sync_copy(acc_sh.at[pl.ds(off, RCH)], rows0_v)
        pltpu.sync_copy(rows0_v, sum_out.at[c, pl.ds(off, RCH)])
        return carry

    lax.fori_loop(0, nch, writeout, 0)


def _wait_idx(src_hbm, sidx, didx, semi):
    pltpu.make_async_copy(src_hbm.at[pl.ds(0, SEG)], sidx, semi).wait()
    pltpu.make_async_copy(src_hbm.at[pl.ds(0, SEG)], didx, semi).wait()


def _make_seg_sum(width):
    return pl.kernel(
        functools.partial(_seg_sum_body, width),
        out_type=[jax.ShapeDtypeStruct((NC, N, width), jnp.float32)],
        mesh=_mesh,
        compiler_params=pltpu.CompilerParams(use_tc_tiling_on_sc=False),
        scratch_types=[
            pltpu.VMEM((SEG, CH), jnp.int32),
            pltpu.VMEM((SEG, CH), jnp.int32),
            pltpu.VMEM((SEG, CH), jnp.int32),
            pltpu.VMEM((SEG, CH), jnp.int32),
            pltpu.VMEM((CH, width), jnp.float32),
            pltpu.VMEM((CH, width), jnp.float32),
            pltpu.VMEM_SHARED((N, width), jnp.float32),
            pltpu.SemaphoreType.DMA,
            pltpu.SemaphoreType.DMA,
            pltpu.SemaphoreType.DMA,
        ],
    )


_seg_sum_aug = _make_seg_sum(DA)
_seg_sum_plain = _make_seg_sum(D)


TB = 1000
GRID = N // TB


def _tc1_body(s_ref, x_ref, wn_ref, ws_ref, b_ref, o_ref, r_ref):
    ssum = s_ref[0] + s_ref[1]
    cnt = ssum[:, D:D + 1]
    recip = 1.0 / jnp.maximum(cnt, 1.0)
    mean = ssum[:, :D] * recip
    y = (jnp.dot(mean, wn_ref[...], preferred_element_type=jnp.float32)
         + jnp.dot(x_ref[...], ws_ref[...], preferred_element_type=jnp.float32)
         + b_ref[...])
    o_ref[...] = jnp.maximum(y, 0.0)
    r_ref[...] = jnp.broadcast_to(recip, (TB, 8))


def _tc2_body(s_ref, r_ref, h_ref, wn_ref, ws_ref, b_ref, o_ref):
    ssum = s_ref[0] + s_ref[1]
    mean = ssum * r_ref[:, 0:1]
    o_ref[...] = (
        jnp.dot(mean, wn_ref[...], preferred_element_type=jnp.float32)
        + jnp.dot(h_ref[...], ws_ref[...], preferred_element_type=jnp.float32)
        + b_ref[...])


_tc1 = pl.pallas_call(
    _tc1_body,
    grid=(GRID,),
    in_specs=[
        pl.BlockSpec((NC, TB, DA), lambda i: (0, i, 0)),
        pl.BlockSpec((TB, D), lambda i: (i, 0)),
        pl.BlockSpec((D, D), lambda i: (0, 0)),
        pl.BlockSpec((D, D), lambda i: (0, 0)),
        pl.BlockSpec((1, D), lambda i: (0, 0)),
    ],
    out_specs=[
        pl.BlockSpec((TB, D), lambda i: (i, 0)),
        pl.BlockSpec((TB, 8), lambda i: (i, 0)),
    ],
    out_shape=[
        jax.ShapeDtypeStruct((N, D), jnp.float32),
        jax.ShapeDtypeStruct((N, 8), jnp.float32),
    ],
)

_tc2 = pl.pallas_call(
    _tc2_body,
    grid=(GRID,),
    in_specs=[
        pl.BlockSpec((NC, TB, D), lambda i: (0, i, 0)),
        pl.BlockSpec((TB, 8), lambda i: (i, 0)),
        pl.BlockSpec((TB, D), lambda i: (i, 0)),
        pl.BlockSpec((D, D), lambda i: (0, 0)),
        pl.BlockSpec((D, D), lambda i: (0, 0)),
        pl.BlockSpec((1, D), lambda i: (0, 0)),
    ],
    out_specs=pl.BlockSpec((TB, D), lambda i: (i, 0)),
    out_shape=jax.ShapeDtypeStruct((N, D), jnp.float32),
)


def kernel(node_features, edge_index, W1_neigh, W1_self, b1,
           W2_neigh, W2_self, b2):
    src = edge_index[0].reshape(E // CH, CH)
    dst = edge_index[1].reshape(E // CH, CH)
    ones_pad = jnp.concatenate(
        [jnp.ones((N, 1), jnp.float32), jnp.zeros((N, DA - D - 1), jnp.float32)],
        axis=1)
    x_aug = jnp.concatenate([node_features, ones_pad], axis=1)
    zrow_a = jnp.zeros((RCH, DA), jnp.float32)
    zrow = jnp.zeros((RCH, D), jnp.float32)

    (sums1,) = _seg_sum_aug(src, dst, x_aug, zrow_a)
    h, rcp = _tc1(sums1, node_features, W1_neigh.T, W1_self.T,
                  b1.reshape(1, D))
    (sums2,) = _seg_sum_plain(src, dst, h, zrow)
    out = _tc2(sums2, rcp, h, W2_neigh.T, W2_self.T, b2.reshape(1, D))
    return out


# speedup vs baseline: 1.0965x; 1.0226x over previous
import functools

import jax
import jax.numpy as jnp
from jax import lax
from jax.experimental import pallas as pl
from jax.experimental.pallas import tpu as pltpu
from jax.experimental.pallas import tpu_sc as plsc

N = 10000
D = 128
DA = 144
E = 320000
NC = 2
NS = 16
NW = NC * NS
EPW = E // NW
CH = 80
NCHUNK = EPW // CH
SEG = 25
RPT = 640
RCH = 80

_mesh = plsc.VectorSubcoreMesh(core_axis_name="c", subcore_axis_name="s")


def _seg_sum_body(width, *refs):
    (src_hbm, dst_hbm, feat_hbm, zrow_hbm, sum_out,
     sidx0_v, didx0_v, sidx1_v, didx1_v, rows0_v, rows1_v, acc_sh,
     sem0, sem1, semi) = refs

    c = lax.axis_index("c")
    s = lax.axis_index("s")
    wid = c * NS + s

    r0 = s * RPT
    nch = jnp.where(s < NS - 1, RPT // RCH, (N - (NS - 1) * RPT) // RCH)

    pltpu.async_copy(src_hbm.at[pl.ds(wid * NCHUNK, SEG)], sidx0_v, semi)
    pltpu.async_copy(dst_hbm.at[pl.ds(wid * NCHUNK, SEG)], didx0_v, semi)

    pltpu.sync_copy(zrow_hbm, rows0_v)

    def zinit(i, carry):
        pltpu.sync_copy(rows0_v, acc_sh.at[pl.ds(r0 + i * RCH, RCH)])
        return carry

    lax.fori_loop(0, nch, zinit, 0)
    _wait_idx(src_hbm, sidx0_v, didx0_v, semi)
    plsc.subcore_barrier()

    bufs = (rows0_v, rows1_v)
    sems = (sem0, sem1)
    idxb = ((sidx0_v, didx0_v), (sidx1_v, didx1_v))
    NSEG = NCHUNK // SEG

    def start_gather(sidx, k, b):
        pltpu.async_copy(feat_hbm.at[sidx.at[k]], bufs[b], sems[b])

    def wait_gather(sidx, b):
        pltpu.make_async_copy(feat_hbm.at[sidx.at[0]], bufs[b], sems[b]).wait()

    def scatter(didx, k, b):
        pltpu.sync_copy(bufs[b], acc_sh.at[didx.at[k]], add=True)

    start_gather(idxb[0][0], 0, 0)
    for k in range(NCHUNK):
        b = k % 2
        seg, loc = divmod(k, SEG)
        sidx, didx = idxb[seg % 2]
        if loc == 0 and seg + 1 < NSEG:
            nsidx, ndidx = idxb[(seg + 1) % 2]
            row = wid * NCHUNK + (seg + 1) * SEG
            pltpu.async_copy(src_hbm.at[pl.ds(row, SEG)], nsidx, semi)
            pltpu.async_copy(dst_hbm.at[pl.ds(row, SEG)], ndidx, semi)
        if k + 1 < NCHUNK:
            seg2, loc2 = divmod(k + 1, SEG)
            s2, d2 = idxb[seg2 % 2]
            if loc2 == 0:
                _wait_idx(src_hbm, s2, d2, semi)
            start_gather(s2, loc2, 1 - b)
        wait_gather(sidx, b)
        scatter(didx, loc, b)

    plsc.subcore_barrier()

    def writeout(i, carry):
        off = r0 + i * RCH
        pltpu.sync_copy(acc_sh.at[pl.ds(off, RCH)], rows0_v)
        pltpu.sync_copy(rows0_v, sum_out.at[c, pl.ds(off, RCH)])
        return carry

    lax.fori_loop(0, nch, writeout, 0)


def _wait_idx(src_hbm, sidx, didx, semi):
    pltpu.make_async_copy(src_hbm.at[pl.ds(0, SEG)], sidx, semi).wait()
    pltpu.make_async_copy(src_hbm.at[pl.ds(0, SEG)], didx, semi).wait()


def _make_seg_sum(width):
    return pl.kernel(
        functools.partial(_seg_sum_body, width),
        out_type=[jax.ShapeDtypeStruct((NC, N, width), jnp.float32)],
        mesh=_mesh,
        compiler_params=pltpu.CompilerParams(use_tc_tiling_on_sc=False),
        scratch_types=[
            pltpu.VMEM((SEG, CH), jnp.int32),
            pltpu.VMEM((SEG, CH), jnp.int32),
            pltpu.VMEM((SEG, CH), jnp.int32),
            pltpu.VMEM((SEG, CH), jnp.int32),
            pltpu.VMEM((CH, width), jnp.float32),
            pltpu.VMEM((CH, width), jnp.float32),
            pltpu.VMEM_SHARED((N, width), jnp.float32),
            pltpu.SemaphoreType.DMA,
            pltpu.SemaphoreType.DMA,
            pltpu.SemaphoreType.DMA,
        ],
    )


_seg_sum_aug = _make_seg_sum(DA)
_seg_sum_plain = _make_seg_sum(D)


TB = 1000
GRID = N // TB


def _tc1_body(s_ref, x_ref, wn_ref, ws_ref, b_ref, o_ref, r_ref):
    ssum = s_ref[0] + s_ref[1]
    cnt = ssum[:, D:D + 1]
    recip = 1.0 / jnp.maximum(cnt, 1.0)
    mean = ssum[:, :D] * recip
    y = (jnp.dot(mean, wn_ref[...], preferred_element_type=jnp.float32)
         + jnp.dot(x_ref[...], ws_ref[...], preferred_element_type=jnp.float32)
         + b_ref[...])
    o_ref[...] = jnp.maximum(y, 0.0)
    r_ref[...] = jnp.broadcast_to(recip, (TB, 8))


def _tc2_body(s_ref, r_ref, h_ref, wn_ref, ws_ref, b_ref, o_ref):
    ssum = s_ref[0] + s_ref[1]
    mean = ssum * r_ref[:, 0:1]
    o_ref[...] = (
        jnp.dot(mean, wn_ref[...], preferred_element_type=jnp.float32)
        + jnp.dot(h_ref[...], ws_ref[...], preferred_element_type=jnp.float32)
        + b_ref[...])


_tc1 = pl.pallas_call(
    _tc1_body,
    grid=(GRID,),
    in_specs=[
        pl.BlockSpec((NC, TB, DA), lambda i: (0, i, 0)),
        pl.BlockSpec((TB, D), lambda i: (i, 0)),
        pl.BlockSpec((D, D), lambda i: (0, 0)),
        pl.BlockSpec((D, D), lambda i: (0, 0)),
        pl.BlockSpec((1, D), lambda i: (0, 0)),
    ],
    out_specs=[
        pl.BlockSpec((TB, D), lambda i: (i, 0)),
        pl.BlockSpec((TB, 8), lambda i: (i, 0)),
    ],
    out_shape=[
        jax.ShapeDtypeStruct((N, D), jnp.float32),
        jax.ShapeDtypeStruct((N, 8), jnp.float32),
    ],
)

_tc2 = pl.pallas_call(
    _tc2_body,
    grid=(GRID,),
    in_specs=[
        pl.BlockSpec((NC, TB, D), lambda i: (0, i, 0)),
        pl.BlockSpec((TB, 8), lambda i: (i, 0)),
        pl.BlockSpec((TB, D), lambda i: (i, 0)),
        pl.BlockSpec((D, D), lambda i: (0, 0)),
        pl.BlockSpec((D, D), lambda i: (0, 0)),
        pl.BlockSpec((1, D), lambda i: (0, 0)),
    ],
    out_specs=pl.BlockSpec((TB, D), lambda i: (i, 0)),
    out_shape=jax.ShapeDtypeStruct((N, D), jnp.float32),
)


def kernel(node_features, edge_index, W1_neigh, W1_self, b1,
           W2_neigh, W2_self, b2):
    src = edge_index[0].reshape(E // CH, CH)
    dst = edge_index[1].reshape(E // CH, CH)
    ones_pad = jnp.concatenate(
        [jnp.ones((N, 1), jnp.float32), jnp.zeros((N, DA - D - 1), jnp.float32)],
        axis=1)
    x_aug = jnp.concatenate([node_features, ones_pad], axis=1)
    zrow_a = jnp.zeros((RCH, DA), jnp.float32)
    zrow = jnp.zeros((RCH, D), jnp.float32)

    (sums1,) = _seg_sum_aug(src, dst, x_aug, zrow_a)
    h, rcp = _tc1(sums1, node_features, W1_neigh.T, W1_self.T,
                  b1.reshape(1, D))
    (sums2,) = _seg_sum_plain(src, dst, h, zrow)
    out = _tc2(sums2, rcp, h, W2_neigh.T, W2_self.T, b2.reshape(1, D))
    return out
